# baseline (speedup 1.0000x reference)
import functools

import jax
import jax.numpy as jnp
from jax import lax
from jax.experimental import pallas as pl
from jax.experimental.pallas import tpu as pltpu
from jax.experimental.pallas import tpu_sc as plsc

EMB_D = 768
KEY_D = 768
POOL = 100
P_LEN = 8
TOP_K = 5
B = 32
CLS = 50
TOK = 77
PREFIX = 1
MID = P_LEN * TOP_K
ROW_D = P_LEN * EMB_D
IDX_PAD = 128
GATH = 8


def _l2n(x):
    n = jnp.linalg.norm(x, axis=1, keepdims=True)
    return x / jnp.clip(n, 1e-12)


def _topk_body(q_ref, k_ref, idx_ref):
    s = lax.dot_general(
        q_ref[:], k_ref[:], (((1,), (1,)), ((), ())),
        preferred_element_type=jnp.float32,
    )
    col = lax.broadcasted_iota(jnp.int32, s.shape, 1)
    parts = []
    for _ in range(TOP_K):
        m = jnp.max(s, axis=1, keepdims=True)
        amax = jnp.min(jnp.where(s == m, col, POOL), axis=1)
        parts.append(amax[:, None])
        s = jnp.where(col == amax[:, None], -jnp.inf, s)
    parts.append(jnp.zeros((B, IDX_PAD - TOP_K), jnp.int32))
    idx_ref[:] = jnp.concatenate(parts, axis=1)


CLS_BLK = 10
CLS_GRID = CLS // CLS_BLK


def _assemble_body(xb_ref, sel_ref, out_ref):
    base = pl.program_id(1) * CLS_BLK
    out_ref[:, 0:PREFIX, :] = xb_ref[pl.ds(base, CLS_BLK), 0:PREFIX, :]
    out_ref[:, PREFIX:PREFIX + MID, :] = jnp.broadcast_to(
        sel_ref[:], (CLS_BLK, MID, EMB_D))
    out_ref[:, PREFIX + MID:, :] = xb_ref[pl.ds(base, CLS_BLK), PREFIX + MID:, :]


def kernel(x_querry, x_block, prompt_tokens, key_tokens):
    n_k = _l2n(key_tokens)
    q_n = lax.stop_gradient(_l2n(x_querry))
    k_idx = pl.pallas_call(
        _topk_body,
        out_shape=jax.ShapeDtypeStruct((B, IDX_PAD), jnp.int32),
    )(q_n, n_k)

    info = plsc.get_sparse_core_info()
    nc, ns = info.num_cores, info.num_subcores

    mesh = plsc.VectorSubcoreMesh(core_axis_name="c", subcore_axis_name="s")

    @functools.partial(
        pl.kernel,
        out_type=jax.ShapeDtypeStruct((B, GATH, ROW_D), jnp.float32),
        mesh=mesh,
        scratch_types=[
            pltpu.VMEM((GATH,), jnp.int32),
            pltpu.VMEM((GATH, ROW_D), jnp.float32),
            pltpu.SemaphoreType.DMA,
        ],
    )
    def _gather_sel(idx_hbm, prompt_hbm, out_hbm, idx8_v, rows_v, sem):
        b = lax.axis_index("s") * nc + lax.axis_index("c")
        pltpu.sync_copy(idx_hbm.at[b, pl.ds(0, GATH)], idx8_v)
        pltpu.async_copy(prompt_hbm.at[idx8_v], rows_v, sem).wait()
        pltpu.sync_copy(rows_v, out_hbm.at[b])

    sel = _gather_sel(k_idx, prompt_tokens.reshape(POOL, ROW_D))

    out = pl.pallas_call(
        _assemble_body,
        grid=(B, CLS_GRID),
        in_specs=[
            pl.BlockSpec((CLS, TOK, EMB_D), lambda b, j: (0, 0, 0)),
            pl.BlockSpec((1, MID, EMB_D), lambda b, j: (b, 0, 0)),
        ],
        out_specs=pl.BlockSpec(
            (CLS_BLK, TOK, EMB_D), lambda b, j: (b * CLS_GRID + j, 0, 0)),
        out_shape=jax.ShapeDtypeStruct((B * CLS, TOK, EMB_D), jnp.float32),
    )(x_block, sel.reshape(B, GATH * P_LEN, EMB_D))
    return out


# speedup vs baseline: 1.0745x; 1.0745x over previous
import functools

import jax
import jax.numpy as jnp
from jax import lax
from jax.experimental import pallas as pl
from jax.experimental.pallas import tpu as pltpu
from jax.experimental.pallas import tpu_sc as plsc

EMB_D = 768
KEY_D = 768
POOL = 100
P_LEN = 8
TOP_K = 5
B = 32
CLS = 50
TOK = 77
PREFIX = 1
MID = P_LEN * TOP_K
ROW_D = P_LEN * EMB_D
IDX_PAD = 128
GATH = 8


def _l2n(x):
    n = jnp.linalg.norm(x, axis=1, keepdims=True)
    return x / jnp.clip(n, 1e-12)


def _topk_body(q_ref, k_ref, idx_ref):
    s = lax.dot_general(
        q_ref[:], k_ref[:], (((1,), (1,)), ((), ())),
        preferred_element_type=jnp.float32,
    )
    col = lax.broadcasted_iota(jnp.int32, s.shape, 1)
    parts = []
    for _ in range(TOP_K):
        m = jnp.max(s, axis=1, keepdims=True)
        amax = jnp.min(jnp.where(s == m, col, POOL), axis=1)
        parts.append(amax[:, None])
        s = jnp.where(col == amax[:, None], -jnp.inf, s)
    parts.append(jnp.zeros((B, IDX_PAD - TOP_K), jnp.int32))
    idx_ref[:] = jnp.concatenate(parts, axis=1)


CLS_BLK = 25
CLS_GRID = CLS // CLS_BLK
NBUF = 4
N_STEPS = B * CLS_GRID


def _assemble_body(xb_ref, sel_ref, out_ref, buf_ref, sem):
    step = pl.program_id(0) * CLS_GRID + pl.program_id(1)
    slot = lax.rem(step, NBUF)
    base = pl.program_id(1) * CLS_BLK
    buf = buf_ref.at[slot]

    @pl.when(step >= NBUF)
    def _wait_slot():
        pltpu.make_async_copy(
            buf, out_ref.at[pl.ds(0, CLS_BLK)], sem.at[slot]).wait()

    buf[:, 0:PREFIX, :] = xb_ref[pl.ds(base, CLS_BLK), 0:PREFIX, :]
    buf[:, PREFIX:PREFIX + MID, :] = jnp.broadcast_to(
        sel_ref[:], (CLS_BLK, MID, EMB_D))
    buf[:, PREFIX + MID:, :] = xb_ref[pl.ds(base, CLS_BLK), PREFIX + MID:, :]

    pltpu.make_async_copy(
        buf, out_ref.at[pl.ds(step * CLS_BLK, CLS_BLK)], sem.at[slot]).start()

    @pl.when(step == N_STEPS - 1)
    def _drain():
        for s in range(NBUF):
            pltpu.make_async_copy(
                buf_ref.at[s], out_ref.at[pl.ds(0, CLS_BLK)], sem.at[s]).wait()


def kernel(x_querry, x_block, prompt_tokens, key_tokens):
    n_k = _l2n(key_tokens)
    q_n = lax.stop_gradient(_l2n(x_querry))
    k_idx = pl.pallas_call(
        _topk_body,
        out_shape=jax.ShapeDtypeStruct((B, IDX_PAD), jnp.int32),
    )(q_n, n_k)

    info = plsc.get_sparse_core_info()
    nc, ns = info.num_cores, info.num_subcores

    mesh = plsc.VectorSubcoreMesh(core_axis_name="c", subcore_axis_name="s")

    @functools.partial(
        pl.kernel,
        out_type=jax.ShapeDtypeStruct((B, GATH, ROW_D), jnp.float32),
        mesh=mesh,
        scratch_types=[
            pltpu.VMEM((GATH,), jnp.int32),
            pltpu.VMEM((GATH, ROW_D), jnp.float32),
            pltpu.SemaphoreType.DMA,
        ],
    )
    def _gather_sel(idx_hbm, prompt_hbm, out_hbm, idx8_v, rows_v, sem):
        b = lax.axis_index("s") * nc + lax.axis_index("c")
        pltpu.sync_copy(idx_hbm.at[b, pl.ds(0, GATH)], idx8_v)
        pltpu.async_copy(prompt_hbm.at[idx8_v], rows_v, sem).wait()
        pltpu.sync_copy(rows_v, out_hbm.at[b])

    sel = _gather_sel(k_idx, prompt_tokens.reshape(POOL, ROW_D))

    out = pl.pallas_call(
        _assemble_body,
        grid=(B, CLS_GRID),
        in_specs=[
            pl.BlockSpec((CLS, TOK, EMB_D), lambda b, j: (0, 0, 0)),
            pl.BlockSpec((1, MID, EMB_D), lambda b, j: (b, 0, 0)),
        ],
        out_specs=pl.BlockSpec(memory_space=pl.ANY),
        out_shape=jax.ShapeDtypeStruct((B * CLS, TOK, EMB_D), jnp.float32),
        scratch_shapes=[
            pltpu.VMEM((NBUF, CLS_BLK, TOK, EMB_D), jnp.float32),
            pltpu.SemaphoreType.DMA((NBUF,)),
        ],
    )(x_block, sel.reshape(B, GATH * P_LEN, EMB_D))
    return out


# speedup vs baseline: 2.8415x; 2.6445x over previous
import functools

import jax
import jax.numpy as jnp
from jax import lax
from jax.experimental import pallas as pl
from jax.experimental.pallas import tpu as pltpu
from jax.experimental.pallas import tpu_sc as plsc

EMB_D = 768
KEY_D = 768
POOL = 100
P_LEN = 8
TOP_K = 5
B = 32
CLS = 50
TOK = 77
PREFIX = 1
MID = P_LEN * TOP_K
ROW_D = P_LEN * EMB_D
IDX_PAD = 128
GATH = 8


def _l2n(x):
    n = jnp.linalg.norm(x, axis=1, keepdims=True)
    return x / jnp.clip(n, 1e-12)


def _topk_body(q_ref, k_ref, idx_ref):
    s = lax.dot_general(
        q_ref[:], k_ref[:], (((1,), (1,)), ((), ())),
        preferred_element_type=jnp.float32,
    )
    col = lax.broadcasted_iota(jnp.int32, s.shape, 1)
    parts = []
    for _ in range(TOP_K):
        m = jnp.max(s, axis=1, keepdims=True)
        amax = jnp.min(jnp.where(s == m, col, POOL), axis=1)
        parts.append(amax[:, None])
        s = jnp.where(col == amax[:, None], -jnp.inf, s)
    parts.append(jnp.zeros((B, IDX_PAD - TOP_K), jnp.int32))
    idx_ref[:] = jnp.concatenate(parts, axis=1)


def _assemble_body(xb_ref, sel_ref, out_ref):
    t = pl.program_id(0)
    mid = jnp.logical_and(t >= PREFIX, t < PREFIX + MID)

    @pl.when(mid)
    def _mid():
        v = sel_ref[0]
        for b in range(B):
            out_ref[0, b * CLS:(b + 1) * CLS, :] = jnp.broadcast_to(
                v[b][None, :], (CLS, EMB_D))

    @pl.when(jnp.logical_not(mid))
    def _xb():
        slab = xb_ref[:, t, :]
        for b in range(B):
            out_ref[0, b * CLS:(b + 1) * CLS, :] = slab


def kernel(x_querry, x_block, prompt_tokens, key_tokens):
    n_k = _l2n(key_tokens)
    q_n = lax.stop_gradient(_l2n(x_querry))
    k_idx = pl.pallas_call(
        _topk_body,
        out_shape=jax.ShapeDtypeStruct((B, IDX_PAD), jnp.int32),
    )(q_n, n_k)

    info = plsc.get_sparse_core_info()
    nc, ns = info.num_cores, info.num_subcores

    mesh = plsc.VectorSubcoreMesh(core_axis_name="c", subcore_axis_name="s")

    @functools.partial(
        pl.kernel,
        out_type=jax.ShapeDtypeStruct((B, GATH, ROW_D), jnp.float32),
        mesh=mesh,
        scratch_types=[
            pltpu.VMEM((GATH,), jnp.int32),
            pltpu.VMEM((GATH, ROW_D), jnp.float32),
            pltpu.SemaphoreType.DMA,
        ],
    )
    def _gather_sel(idx_hbm, prompt_hbm, out_hbm, idx8_v, rows_v, sem):
        b = lax.axis_index("s") * nc + lax.axis_index("c")
        pltpu.sync_copy(idx_hbm.at[b, pl.ds(0, GATH)], idx8_v)
        pltpu.async_copy(prompt_hbm.at[idx8_v], rows_v, sem).wait()
        pltpu.sync_copy(rows_v, out_hbm.at[b])

    sel = _gather_sel(k_idx, prompt_tokens.reshape(POOL, ROW_D))

    sel_t = sel.reshape(B, GATH * P_LEN, EMB_D)[:, :MID, :].transpose(1, 0, 2)
    out77 = pl.pallas_call(
        _assemble_body,
        grid=(TOK,),
        in_specs=[
            pl.BlockSpec((CLS, TOK, EMB_D), lambda t: (0, 0, 0)),
            pl.BlockSpec(
                (1, B, EMB_D),
                lambda t: (jnp.clip(t - PREFIX, 0, MID - 1), 0, 0)),
        ],
        out_specs=pl.BlockSpec((1, B * CLS, EMB_D), lambda t: (t, 0, 0)),
        out_shape=jax.ShapeDtypeStruct((TOK, B * CLS, EMB_D), jnp.float32),
    )(x_block, sel_t)
    return out77.transpose(1, 0, 2)


# speedup vs baseline: 2.9856x; 1.0507x over previous
import functools

import jax
import jax.numpy as jnp
from jax import lax
from jax.experimental import pallas as pl
from jax.experimental.pallas import tpu as pltpu
from jax.experimental.pallas import tpu_sc as plsc

EMB_D = 768
KEY_D = 768
POOL = 100
P_LEN = 8
TOP_K = 5
B = 32
CLS = 50
TOK = 77
PREFIX = 1
MID = P_LEN * TOP_K
ROW_D = P_LEN * EMB_D
IDX_PAD = 128
GATH = 8


def _l2n(x):
    n = jnp.linalg.norm(x, axis=1, keepdims=True)
    return x / jnp.clip(n, 1e-12)


def _topk_body(q_ref, k_ref, idx_ref):
    s = lax.dot_general(
        q_ref[:], k_ref[:], (((1,), (1,)), ((), ())),
        preferred_element_type=jnp.float32,
    )
    col = lax.broadcasted_iota(jnp.int32, s.shape, 1)
    parts = []
    for _ in range(TOP_K):
        m = jnp.max(s, axis=1, keepdims=True)
        amax = jnp.min(jnp.where(s == m, col, POOL), axis=1)
        parts.append(amax[:, None])
        s = jnp.where(col == amax[:, None], -jnp.inf, s)
    parts.append(jnp.zeros((B, IDX_PAD - TOP_K), jnp.int32))
    idx_ref[:] = jnp.concatenate(parts, axis=1)


def _asm_xb_body(xb_ref, out_ref):
    i = pl.program_id(0)
    t = jnp.where(i == 0, 0, i + MID)
    slab = xb_ref[:, t, :]
    for b in range(B):
        out_ref[0, b * CLS:(b + 1) * CLS, :] = slab


def _asm_sel_body(prev_ref, sel_ref, out_ref):
    del prev_ref
    j = pl.program_id(0)
    v = sel_ref[:, j, :]
    for b in range(B):
        out_ref[0, b * CLS:(b + 1) * CLS, :] = jnp.broadcast_to(
            v[b][None, :], (CLS, EMB_D))


def kernel(x_querry, x_block, prompt_tokens, key_tokens):
    n_k = _l2n(key_tokens)
    q_n = lax.stop_gradient(_l2n(x_querry))
    k_idx = pl.pallas_call(
        _topk_body,
        out_shape=jax.ShapeDtypeStruct((B, IDX_PAD), jnp.int32),
    )(q_n, n_k)

    info = plsc.get_sparse_core_info()
    nc, ns = info.num_cores, info.num_subcores

    mesh = plsc.VectorSubcoreMesh(core_axis_name="c", subcore_axis_name="s")

    @functools.partial(
        pl.kernel,
        out_type=jax.ShapeDtypeStruct((B, GATH, ROW_D), jnp.float32),
        mesh=mesh,
        scratch_types=[
            pltpu.VMEM((GATH,), jnp.int32),
            pltpu.VMEM((GATH, ROW_D), jnp.float32),
            pltpu.SemaphoreType.DMA,
        ],
    )
    def _gather_sel(idx_hbm, prompt_hbm, out_hbm, idx8_v, rows_v, sem):
        b = lax.axis_index("s") * nc + lax.axis_index("c")
        pltpu.sync_copy(idx_hbm.at[b, pl.ds(0, GATH)], idx8_v)
        pltpu.async_copy(prompt_hbm.at[idx8_v], rows_v, sem).wait()
        pltpu.sync_copy(rows_v, out_hbm.at[b])

    sel = _gather_sel(k_idx, prompt_tokens.reshape(POOL, ROW_D))

    out_shape = jax.ShapeDtypeStruct((TOK, B * CLS, EMB_D), jnp.float32)
    out_xb = pl.pallas_call(
        _asm_xb_body,
        grid=(TOK - MID,),
        in_specs=[pl.BlockSpec((CLS, TOK, EMB_D), lambda i: (0, 0, 0))],
        out_specs=pl.BlockSpec(
            (1, B * CLS, EMB_D),
            lambda i: (jnp.where(i == 0, 0, i + MID), 0, 0)),
        out_shape=out_shape,
    )(x_block)
    out77 = pl.pallas_call(
        _asm_sel_body,
        grid=(MID,),
        in_specs=[
            pl.BlockSpec(memory_space=pl.ANY),
            pl.BlockSpec((B, GATH * P_LEN, EMB_D), lambda j: (0, 0, 0)),
        ],
        out_specs=pl.BlockSpec(
            (1, B * CLS, EMB_D), lambda j: (j + PREFIX, 0, 0)),
        out_shape=out_shape,
        input_output_aliases={0: 0},
    )(out_xb, sel.reshape(B, GATH * P_LEN, EMB_D))
    return out77.transpose(1, 0, 2)
